# baseline (speedup 1.0000x reference)
import functools

import jax
import jax.numpy as jnp
from jax import lax
from jax.experimental import pallas as pl
from jax.experimental.pallas import tpu as pltpu
from jax.experimental.pallas import tpu_sc as plsc

N_NODES = 10000
IN_CH = 128
HIDDEN = 128
OUT_CH = 8
OUT_PAD = 16

NC = 2
NS = 16
NW = NC * NS
EPW = 10240
E_PAD = NW * EPW
N_PAD = 10112
RPT = N_PAD // NS

_MESH = plsc.VectorSubcoreMesh(core_axis_name="c", subcore_axis_name="s")


_SC_PARAMS = pltpu.CompilerParams(use_tc_tiling_on_sc=False)
CH = 64
N0CH = 172
N1CH = 142
NCHMX = max(N0CH, N1CH)
E0 = NS * N0CH * CH
E1 = NS * N1CH * CH


def _make_agg(width):
  @functools.partial(
      pl.kernel,
      out_type=jax.ShapeDtypeStruct((NC, N_PAD, width), jnp.float32),
      mesh=_MESH,
      compiler_params=_SC_PARAMS,
      scratch_types=[
          pltpu.VMEM((NCHMX * CH,), jnp.int32),
          pltpu.VMEM((NCHMX, CH), jnp.int32),
          pltpu.VMEM((CH, width), jnp.float32),
          pltpu.VMEM((CH, width), jnp.float32),
          pltpu.VMEM_SHARED((N_PAD, width), jnp.float32),
          pltpu.SemaphoreType.DMA,
          pltpu.SemaphoreType.DMA,
      ],
  )
  def agg(table, zeros, src_hbm, dst_hbm, out, src_v, dst_v, buf0, buf1,
          acc, sem0, sem1):
    cid = lax.axis_index("c")
    sid = lax.axis_index("s")
    wid = cid * NS + sid
    rows = pl.ds(sid * RPT, RPT)
    nch = jnp.where(cid == 0, N0CH, N1CH)

    pltpu.sync_copy(src_hbm.at[wid], src_v)
    pltpu.sync_copy(dst_hbm.at[wid], dst_v)

    @pl.when(cid == 0)
    def _():
      pltpu.sync_copy(table.at[rows], acc.at[rows])

    @pl.when(cid != 0)
    def _():
      pltpu.sync_copy(zeros.at[rows], acc.at[rows])

    plsc.subcore_barrier()

    bufs = (buf0, buf1)
    sems = (sem0, sem1)
    pltpu.async_copy(table.at[src_v.at[pl.ds(0, CH)]], buf0, sem0)
    pltpu.async_copy(table.at[src_v.at[pl.ds(CH, CH)]], buf1, sem1)

    @pl.loop(0, nch, step=2)
    def _(j):
      for b in range(2):
        jj = j + b
        pltpu.make_async_copy(
            table.at[src_v.at[pl.ds(jj * CH, CH)]], bufs[b], sems[b]).wait()
        pltpu.sync_copy(bufs[b], acc.at[dst_v.at[jj]], add=True)

        @pl.when(jj + 2 < nch)
        def _():
          pltpu.async_copy(
              table.at[src_v.at[pl.ds((jj + 2) * CH, CH)]], bufs[b], sems[b])

    plsc.subcore_barrier()
    pltpu.sync_copy(acc.at[rows], out.at[cid].at[rows])

  return agg


_agg128 = _make_agg(IN_CH)

CH16 = 128
N0CH16 = 114
N1CH16 = 44
E0_16 = NS * N0CH16 * CH16
E1_16 = NS * N1CH16 * CH16


@functools.partial(
    pl.kernel,
    out_type=jax.ShapeDtypeStruct((NC, N_PAD, OUT_PAD), jnp.float32),
    mesh=_MESH,
    compiler_params=_SC_PARAMS,
    scratch_types=[
        pltpu.VMEM((N0CH16 * CH16,), jnp.int32),
        pltpu.VMEM((N0CH16, CH16), jnp.int32),
        pltpu.VMEM((CH16, OUT_PAD), jnp.float32),
        pltpu.VMEM((CH16, OUT_PAD), jnp.float32),
        pltpu.VMEM_SHARED((N_PAD, OUT_PAD), jnp.float32),
        pltpu.SemaphoreType.DMA,
        pltpu.SemaphoreType.DMA,
    ],
)
def _agg16(table, zeros, src_hbm, dst_hbm, out, src_v, dst_v, buf0, buf1,
           acc, sem0, sem1):
  cid = lax.axis_index("c")
  sid = lax.axis_index("s")
  wid = cid * NS + sid
  rows = pl.ds(sid * RPT, RPT)
  NCH16 = jnp.where(cid == 0, N0CH16, N1CH16)

  pltpu.sync_copy(src_hbm.at[wid], src_v)
  pltpu.sync_copy(dst_hbm.at[wid], dst_v)

  @pl.when(cid == 0)
  def _():
    pltpu.sync_copy(table.at[rows], acc.at[rows])

  @pl.when(cid != 0)
  def _():
    pltpu.sync_copy(zeros.at[rows], acc.at[rows])

  plsc.subcore_barrier()

  bufs = (buf0, buf1)
  sems = (sem0, sem1)
  pltpu.async_copy(table.at[src_v.at[pl.ds(0, CH16)]], buf0, sem0)
  pltpu.async_copy(table.at[src_v.at[pl.ds(CH16, CH16)]], buf1, sem1)

  @pl.loop(0, NCH16, step=2)
  def _(j):
    for b in range(2):
      jj = j + b
      pltpu.make_async_copy(
          table.at[src_v.at[pl.ds(jj * CH16, CH16)]], bufs[b], sems[b]).wait()
      pltpu.sync_copy(bufs[b], acc.at[dst_v.at[jj]], add=True)

      @pl.when(jj + 2 < NCH16)
      def _():
        pltpu.async_copy(
            table.at[src_v.at[pl.ds((jj + 2) * CH16, CH16)]], bufs[b], sems[b])

  plsc.subcore_barrier()
  pltpu.sync_copy(acc.at[rows], out.at[cid].at[rows])


@functools.partial(
    pl.kernel,
    out_type=jax.ShapeDtypeStruct((NC, N_PAD, OUT_PAD), jnp.float32),
    mesh=_MESH,
    compiler_params=_SC_PARAMS,
    scratch_types=[
        pltpu.VMEM((EPW // 128, 128), jnp.int32),
        pltpu.VMEM((128, OUT_PAD), jnp.float32),
        pltpu.VMEM_SHARED((N_PAD, OUT_PAD), jnp.float32),
    ],
)
def _hist(dst_hbm, ones_hbm, zeros_hbm, out, dst_v, ones_v, acc):
  cid = lax.axis_index("c")
  sid = lax.axis_index("s")
  wid = cid * NS + sid
  rows = pl.ds(sid * RPT, RPT)

  pltpu.sync_copy(dst_hbm.at[wid], dst_v)
  pltpu.sync_copy(ones_hbm, ones_v)
  pltpu.sync_copy(zeros_hbm.at[rows], acc.at[rows])
  plsc.subcore_barrier()

  @pl.loop(0, EPW // 128)
  def _(j):
    pltpu.sync_copy(ones_v, acc.at[dst_v.at[j]], add=True)

  plsc.subcore_barrier()
  pltpu.sync_copy(acc.at[rows], out.at[cid].at[rows])


def _tc_prep_body(degp, x, dinv_o, xs_o):
  deg = degp[0, :, 0:1] + degp[1, :, 0:1] + 1.0
  dinv = lax.rsqrt(deg)
  dinv_o[...] = jnp.broadcast_to(dinv, (N_PAD, OUT_PAD))
  xs_o[0:N_NODES] = x[...] * dinv[0:N_NODES]
  xs_o[N_NODES:N_PAD] = jnp.zeros((N_PAD - N_NODES, IN_CH), jnp.float32)


_tc_prep = pl.pallas_call(
    _tc_prep_body,
    out_shape=(
        jax.ShapeDtypeStruct((N_PAD, OUT_PAD), jnp.float32),
        jax.ShapeDtypeStruct((N_PAD, IN_CH), jnp.float32),
    ),
)


def _tc_mid_body(agg1, dinv, W1, b1, W2p, gs_o):
  dcol = dinv[:, 0:1]
  out1 = (agg1[0] + agg1[1]) * dcol
  h = jnp.maximum(
      jnp.dot(out1, W1[...], preferred_element_type=jnp.float32) + b1[...], 0.0)
  g = jnp.dot(h, W2p[...], preferred_element_type=jnp.float32)
  gs_o[...] = g * dcol


_tc_mid = pl.pallas_call(
    _tc_mid_body,
    out_shape=jax.ShapeDtypeStruct((N_PAD, OUT_PAD), jnp.float32),
)


def _tc_final_body(agg2, dinv, b2p, z_o):
  out2 = (agg2[0] + agg2[1]) * dinv[:, 0:1] + b2p[...]
  z_o[...] = out2[0:N_NODES, 0:OUT_CH]


_tc_final = pl.pallas_call(
    _tc_final_body,
    out_shape=jax.ShapeDtypeStruct((N_NODES, OUT_CH), jnp.float32),
)


@jax.jit
def kernel(x, edge_index, W1, b1, W2, b2):
  ei = edge_index.astype(jnp.int32)
  n_edges = ei.shape[1]
  src, dst = ei[0], ei[1]

  n_extra = E_PAD - n_edges
  dst128 = jnp.concatenate(
      [dst, jnp.full((n_extra,), N_NODES, jnp.int32)]
  ).reshape(NW, EPW // 128, 128)

  pad16 = E0_16 + E1_16 - n_edges
  s16b = jnp.concatenate(
      [src[E0_16:], jnp.zeros((pad16,), jnp.int32)]).reshape(NS, N1CH16 * CH16)
  d16b = jnp.concatenate(
      [dst[E0_16:], jnp.full((pad16,), N_NODES, jnp.int32)]
  ).reshape(NS, N1CH16 * CH16)

  def _padw16(a, fill):
    return jnp.pad(a, ((0, 0), (0, (N0CH16 - N1CH16) * CH16)),
                   constant_values=fill)

  src16 = jnp.concatenate(
      [src[:E0_16].reshape(NS, N0CH16 * CH16), _padw16(s16b, 0)], axis=0)
  dst16 = jnp.concatenate(
      [dst[:E0_16].reshape(NS, N0CH16 * CH16), _padw16(d16b, N_NODES)],
      axis=0).reshape(NW, N0CH16, CH16)

  pad1 = E0 + E1 - n_edges
  srcA = src[:E0].reshape(NS, N0CH * CH)
  dstA = dst[:E0].reshape(NS, N0CH * CH)
  srcB = jnp.concatenate(
      [src[E0:], jnp.zeros((pad1,), jnp.int32)]).reshape(NS, N1CH * CH)
  dstB = jnp.concatenate(
      [dst[E0:], jnp.full((pad1,), N_NODES, jnp.int32)]).reshape(NS, N1CH * CH)

  def _padw(a, n, fill):
    return jnp.pad(a, ((0, 0), (0, NCHMX * CH - n)), constant_values=fill)

  src_w = jnp.concatenate(
      [_padw(srcA, N0CH * CH, 0), _padw(srcB, N1CH * CH, 0)], axis=0)
  dst_w = jnp.concatenate(
      [_padw(dstA, N0CH * CH, N_NODES), _padw(dstB, N1CH * CH, N_NODES)],
      axis=0).reshape(NW, NCHMX, CH)

  zeros128 = jnp.zeros((N_PAD, IN_CH), jnp.float32)
  zeros16 = jnp.zeros((N_PAD, OUT_PAD), jnp.float32)
  ones16 = jnp.ones((128, OUT_PAD), jnp.float32)
  W2p = jnp.zeros((HIDDEN, OUT_PAD), jnp.float32).at[:, :OUT_CH].set(W2)
  b1r = b1.reshape(1, HIDDEN)
  b2p = jnp.zeros((1, OUT_PAD), jnp.float32).at[0, :OUT_CH].set(b2)

  degp = _hist(dst128, ones16, zeros16)
  dinv, xs = _tc_prep(degp, x)
  agg1 = _agg128(xs, zeros128, src_w, dst_w)
  gs = _tc_mid(agg1, dinv, W1, b1r, W2p)
  agg2 = _agg16(gs, zeros16, src16, dst16)
  return _tc_final(agg2, dinv, b2p)


# speedup vs baseline: 1.0624x; 1.0624x over previous
import functools

import jax
import jax.numpy as jnp
from jax import lax
from jax.experimental import pallas as pl
from jax.experimental.pallas import tpu as pltpu
from jax.experimental.pallas import tpu_sc as plsc

N_NODES = 10000
IN_CH = 128
HIDDEN = 128
OUT_CH = 8
OUT_PAD = 16

NC = 2
NS = 16
NW = NC * NS
EPW = 10240
E_PAD = NW * EPW
N_PAD = 10112
RPT = N_PAD // NS

_MESH = plsc.VectorSubcoreMesh(core_axis_name="c", subcore_axis_name="s")


_SC_PARAMS = pltpu.CompilerParams(use_tc_tiling_on_sc=False)
CH = 64
N0CH = 180
N1CH = 134
NCHMX = max(N0CH, N1CH)
E0 = NS * N0CH * CH
E1 = NS * N1CH * CH


def _make_agg(width):
  @functools.partial(
      pl.kernel,
      out_type=jax.ShapeDtypeStruct((NC, N_PAD, width), jnp.float32),
      mesh=_MESH,
      compiler_params=_SC_PARAMS,
      scratch_types=[
          pltpu.VMEM((NCHMX * CH,), jnp.int32),
          pltpu.VMEM((NCHMX, CH), jnp.int32),
          pltpu.VMEM((CH, width), jnp.float32),
          pltpu.VMEM((CH, width), jnp.float32),
          pltpu.VMEM_SHARED((N_PAD, width), jnp.float32),
          pltpu.SemaphoreType.DMA,
          pltpu.SemaphoreType.DMA,
      ],
  )
  def agg(table, zeros, src_hbm, dst_hbm, out, src_v, dst_v, buf0, buf1,
          acc, sem0, sem1):
    cid = lax.axis_index("c")
    sid = lax.axis_index("s")
    wid = cid * NS + sid
    rows = pl.ds(sid * RPT, RPT)
    nch = jnp.where(cid == 0, N0CH, N1CH)

    pltpu.sync_copy(src_hbm.at[wid], src_v)
    pltpu.sync_copy(dst_hbm.at[wid], dst_v)

    @pl.when(cid == 0)
    def _():
      pltpu.sync_copy(table.at[rows], acc.at[rows])

    @pl.when(cid != 0)
    def _():
      pltpu.sync_copy(zeros.at[rows], acc.at[rows])

    plsc.subcore_barrier()

    bufs = (buf0, buf1)
    sems = (sem0, sem1)
    pltpu.async_copy(table.at[src_v.at[pl.ds(0, CH)]], buf0, sem0)
    pltpu.async_copy(table.at[src_v.at[pl.ds(CH, CH)]], buf1, sem1)

    @pl.loop(0, nch, step=2)
    def _(j):
      for b in range(2):
        jj = j + b
        pltpu.make_async_copy(
            table.at[src_v.at[pl.ds(jj * CH, CH)]], bufs[b], sems[b]).wait()
        pltpu.sync_copy(bufs[b], acc.at[dst_v.at[jj]], add=True)

        @pl.when(jj + 2 < nch)
        def _():
          pltpu.async_copy(
              table.at[src_v.at[pl.ds((jj + 2) * CH, CH)]], bufs[b], sems[b])

    plsc.subcore_barrier()
    pltpu.sync_copy(acc.at[rows], out.at[cid].at[rows])

  return agg


_agg128 = _make_agg(IN_CH)

CH16 = 128
N0CH16 = 114
N1CH16 = 44
E0_16 = NS * N0CH16 * CH16
E1_16 = NS * N1CH16 * CH16


@functools.partial(
    pl.kernel,
    out_type=jax.ShapeDtypeStruct((NC, N_PAD, OUT_PAD), jnp.float32),
    mesh=_MESH,
    compiler_params=_SC_PARAMS,
    scratch_types=[
        pltpu.VMEM((N0CH16 * CH16,), jnp.int32),
        pltpu.VMEM((N0CH16, CH16), jnp.int32),
        pltpu.VMEM((CH16, OUT_PAD), jnp.float32),
        pltpu.VMEM((CH16, OUT_PAD), jnp.float32),
        pltpu.VMEM_SHARED((N_PAD, OUT_PAD), jnp.float32),
        pltpu.SemaphoreType.DMA,
        pltpu.SemaphoreType.DMA,
    ],
)
def _agg16(table, zeros, src_hbm, dst_hbm, out, src_v, dst_v, buf0, buf1,
           acc, sem0, sem1):
  cid = lax.axis_index("c")
  sid = lax.axis_index("s")
  wid = cid * NS + sid
  rows = pl.ds(sid * RPT, RPT)
  NCH16 = jnp.where(cid == 0, N0CH16, N1CH16)

  pltpu.sync_copy(src_hbm.at[wid], src_v)
  pltpu.sync_copy(dst_hbm.at[wid], dst_v)

  @pl.when(cid == 0)
  def _():
    pltpu.sync_copy(table.at[rows], acc.at[rows])

  @pl.when(cid != 0)
  def _():
    pltpu.sync_copy(zeros.at[rows], acc.at[rows])

  plsc.subcore_barrier()

  bufs = (buf0, buf1)
  sems = (sem0, sem1)
  pltpu.async_copy(table.at[src_v.at[pl.ds(0, CH16)]], buf0, sem0)
  pltpu.async_copy(table.at[src_v.at[pl.ds(CH16, CH16)]], buf1, sem1)

  @pl.loop(0, NCH16, step=2)
  def _(j):
    for b in range(2):
      jj = j + b
      pltpu.make_async_copy(
          table.at[src_v.at[pl.ds(jj * CH16, CH16)]], bufs[b], sems[b]).wait()
      pltpu.sync_copy(bufs[b], acc.at[dst_v.at[jj]], add=True)

      @pl.when(jj + 2 < NCH16)
      def _():
        pltpu.async_copy(
            table.at[src_v.at[pl.ds((jj + 2) * CH16, CH16)]], bufs[b], sems[b])

  plsc.subcore_barrier()
  pltpu.sync_copy(acc.at[rows], out.at[cid].at[rows])


@functools.partial(
    pl.kernel,
    out_type=jax.ShapeDtypeStruct((NC, N_PAD, OUT_PAD), jnp.float32),
    mesh=_MESH,
    compiler_params=_SC_PARAMS,
    scratch_types=[
        pltpu.VMEM((EPW // 128, 128), jnp.int32),
        pltpu.VMEM((128, OUT_PAD), jnp.float32),
        pltpu.VMEM_SHARED((N_PAD, OUT_PAD), jnp.float32),
    ],
)
def _hist(dst_hbm, ones_hbm, zeros_hbm, out, dst_v, ones_v, acc):
  cid = lax.axis_index("c")
  sid = lax.axis_index("s")
  wid = cid * NS + sid
  rows = pl.ds(sid * RPT, RPT)

  pltpu.sync_copy(dst_hbm.at[wid], dst_v)
  pltpu.sync_copy(ones_hbm, ones_v)
  pltpu.sync_copy(zeros_hbm.at[rows], acc.at[rows])
  plsc.subcore_barrier()

  @pl.loop(0, EPW // 128)
  def _(j):
    pltpu.sync_copy(ones_v, acc.at[dst_v.at[j]], add=True)

  plsc.subcore_barrier()
  pltpu.sync_copy(acc.at[rows], out.at[cid].at[rows])


def _tc_prep_body(degp, x, dinv_o, xs_o):
  deg = degp[0, :, 0:1] + degp[1, :, 0:1] + 1.0
  dinv = lax.rsqrt(deg)
  dinv_o[...] = jnp.broadcast_to(dinv, (N_PAD, OUT_PAD))
  xs_o[0:N_NODES] = x[...] * dinv[0:N_NODES]
  xs_o[N_NODES:N_PAD] = jnp.zeros((N_PAD - N_NODES, IN_CH), jnp.float32)


_tc_prep = pl.pallas_call(
    _tc_prep_body,
    out_shape=(
        jax.ShapeDtypeStruct((N_PAD, OUT_PAD), jnp.float32),
        jax.ShapeDtypeStruct((N_PAD, IN_CH), jnp.float32),
    ),
)


def _tc_mid_body(agg1, dinv, W1, b1, W2p, gs_o):
  dcol = dinv[:, 0:1]
  out1 = (agg1[0] + agg1[1]) * dcol
  h = jnp.maximum(
      jnp.dot(out1, W1[...], preferred_element_type=jnp.float32) + b1[...], 0.0)
  g = jnp.dot(h, W2p[...], preferred_element_type=jnp.float32)
  gs_o[...] = g * dcol


_tc_mid = pl.pallas_call(
    _tc_mid_body,
    out_shape=jax.ShapeDtypeStruct((N_PAD, OUT_PAD), jnp.float32),
)


def _tc_final_body(agg2, dinv, b2p, z_o):
  out2 = (agg2[0] + agg2[1]) * dinv[:, 0:1] + b2p[...]
  z_o[...] = out2[0:N_NODES, 0:OUT_CH]


_tc_final = pl.pallas_call(
    _tc_final_body,
    out_shape=jax.ShapeDtypeStruct((N_NODES, OUT_CH), jnp.float32),
)


@jax.jit
def kernel(x, edge_index, W1, b1, W2, b2):
  ei = edge_index.astype(jnp.int32)
  n_edges = ei.shape[1]
  src, dst = ei[0], ei[1]

  n_extra = E_PAD - n_edges
  dst128 = jnp.concatenate(
      [dst, jnp.full((n_extra,), N_NODES, jnp.int32)]
  ).reshape(NW, EPW // 128, 128)

  pad16 = E0_16 + E1_16 - n_edges
  s16b = jnp.concatenate(
      [src[E0_16:], jnp.zeros((pad16,), jnp.int32)]).reshape(NS, N1CH16 * CH16)
  d16b = jnp.concatenate(
      [dst[E0_16:], jnp.full((pad16,), N_NODES, jnp.int32)]
  ).reshape(NS, N1CH16 * CH16)

  def _padw16(a, fill):
    return jnp.pad(a, ((0, 0), (0, (N0CH16 - N1CH16) * CH16)),
                   constant_values=fill)

  src16 = jnp.concatenate(
      [src[:E0_16].reshape(NS, N0CH16 * CH16), _padw16(s16b, 0)], axis=0)
  dst16 = jnp.concatenate(
      [dst[:E0_16].reshape(NS, N0CH16 * CH16), _padw16(d16b, N_NODES)],
      axis=0).reshape(NW, N0CH16, CH16)

  pad1 = E0 + E1 - n_edges
  srcA = src[:E0].reshape(NS, N0CH * CH)
  dstA = dst[:E0].reshape(NS, N0CH * CH)
  srcB = jnp.concatenate(
      [src[E0:], jnp.zeros((pad1,), jnp.int32)]).reshape(NS, N1CH * CH)
  dstB = jnp.concatenate(
      [dst[E0:], jnp.full((pad1,), N_NODES, jnp.int32)]).reshape(NS, N1CH * CH)

  def _padw(a, n, fill):
    return jnp.pad(a, ((0, 0), (0, NCHMX * CH - n)), constant_values=fill)

  src_w = jnp.concatenate(
      [_padw(srcA, N0CH * CH, 0), _padw(srcB, N1CH * CH, 0)], axis=0)
  dst_w = jnp.concatenate(
      [_padw(dstA, N0CH * CH, N_NODES), _padw(dstB, N1CH * CH, N_NODES)],
      axis=0).reshape(NW, NCHMX, CH)

  zeros128 = jnp.zeros((N_PAD, IN_CH), jnp.float32)
  zeros16 = jnp.zeros((N_PAD, OUT_PAD), jnp.float32)
  ones16 = jnp.ones((128, OUT_PAD), jnp.float32)
  W2p = jnp.zeros((HIDDEN, OUT_PAD), jnp.float32).at[:, :OUT_CH].set(W2)
  b1r = b1.reshape(1, HIDDEN)
  b2p = jnp.zeros((1, OUT_PAD), jnp.float32).at[0, :OUT_CH].set(b2)

  degp = _hist(dst128, ones16, zeros16)
  dinv, xs = _tc_prep(degp, x)
  agg1 = _agg128(xs, zeros128, src_w, dst_w)
  gs = _tc_mid(agg1, dinv, W1, b1r, W2p)
  agg2 = _agg16(gs, zeros16, src16, dst16)
  return _tc_final(agg2, dinv, b2p)
